# baseline (speedup 1.0000x reference)
import functools

import jax
import jax.numpy as jnp
import numpy as np
from jax.experimental import pallas as pl

N_POINTS = 30000
FEAT_DIM = 128
EXP_DIM = 64
POSE_DIM = 6
N_LMK = 68
POS_FREQ = 4
NEAR = 0.1
FAR = 0.25
DEFORM_SCALE = 0.3
ATTR_SCALE = 0.05

BLK = 600


def _pos_embed_cols(x):
    outs = [x]
    for i in range(POS_FREQ):
        f = 2.0 ** i
        outs.append(jnp.sin(x * f))
        outs.append(jnp.cos(x * f))
    return jnp.concatenate(outs, axis=-1)


def _so3_exp(log_rot, eps=1e-4):
    theta2 = jnp.clip(jnp.sum(log_rot * log_rot, axis=-1), eps)
    theta = jnp.sqrt(theta2)
    fac1 = jnp.sin(theta) / theta
    fac2 = (1.0 - jnp.cos(theta)) / theta2
    x, y, z = log_rot[..., 0], log_rot[..., 1], log_rot[..., 2]
    zz = jnp.zeros_like(x)
    K = jnp.stack([
        jnp.stack([zz, -z, y], axis=-1),
        jnp.stack([z, zz, -x], axis=-1),
        jnp.stack([-y, x, zz], axis=-1),
    ], axis=-2)
    I = jnp.eye(3, dtype=log_rot.dtype)
    return I + fac1[..., None, None] * K + fac2[..., None, None] * (K @ K)


def _body(blk,
          xyz_ref, feat_ref, sp_ref, rp_ref, op_ref,
          lmkT_ref, lmksq_ref, ec_ref, pose_ref, scale_ref, R_ref, RT_ref,
          Wf_ref, Wx_ref, Wech_ref, bech_ref, Wpeh_ref, bpeh_ref,
          Wce2_ref, bce2_ref, Wae2_ref, bae2_ref, Wde2_ref, bde2_ref,
          Wce3_ref, bce3_ref, Wae3_ref, bae3_ref, Wde3_ref, bde3_ref,
          Wcp2_ref, bcp2_ref, Wap2_ref, bap2_ref, Wdp2_ref, bdp2_ref,
          oxyz_ref, ocol_ref, oscl_ref, orot_ref, oopa_ref):
    f32 = jnp.float32
    xyz = xyz_ref[...]
    feat = jnp.tanh(feat_ref[...])

    lmkT = lmkT_ref[...]
    dx = xyz[:, 0:1] - lmkT[0:1, :]
    dy = xyz[:, 1:2] - lmkT[1:2, :]
    dz = xyz[:, 2:3] - lmkT[2:3, :]
    d2 = dx * dx + dy * dy + dz * dz
    dists = jnp.min(d2, axis=1, keepdims=True)
    exp_w = jnp.clip((FAR - dists) / (FAR - NEAR), 0.0, 1.0)

    emb_xyz = jnp.concatenate(
        [_pos_embed_cols(xyz), jnp.zeros((blk, 5), f32)], axis=1)
    emb_pose = _pos_embed_cols(pose_ref[...])

    F1 = jnp.dot(feat, Wf_ref[...], preferred_element_type=f32)
    X1 = jnp.dot(emb_xyz, Wx_ref[...], preferred_element_type=f32)

    ofs_e = jnp.dot(ec_ref[...], Wech_ref[...],
                    preferred_element_type=f32) + bech_ref[...]
    ofs_p = jnp.dot(emb_pose, Wpeh_ref[...],
                    preferred_element_type=f32) + bpeh_ref[...]

    def stk(a, o):
        return jnp.concatenate([a + o[0:1], a + o[1:2]], axis=0)

    relu = lambda v: jnp.maximum(v, 0.0)
    mm = lambda a, w_ref, b_ref: (
        jnp.dot(a, w_ref[...], preferred_element_type=f32) + b_ref[...])

    hc = relu(stk(F1[:, 0:256], ofs_e[:, 0:256]))
    col_e = mm(relu(mm(hc, Wce2_ref, bce2_ref)), Wce3_ref, bce3_ref)
    ha = relu(stk(F1[:, 256:512], ofs_e[:, 256:512]))
    attr_e = mm(relu(mm(ha, Wae2_ref, bae2_ref)), Wae3_ref, bae3_ref)
    hd = relu(stk(X1[:, 0:256], ofs_e[:, 512:768]))
    dxyz_e = jnp.tanh(mm(relu(mm(hd, Wde2_ref, bde2_ref)), Wde3_ref, bde3_ref))

    hpc = relu(stk(F1[:, 512:640], ofs_p[:, 0:128]))
    col_p = mm(hpc, Wcp2_ref, bcp2_ref)
    hpa = relu(stk(F1[:, 640:768], ofs_p[:, 128:256]))
    attr_p = mm(hpa, Wap2_ref, bap2_ref)
    hpd = relu(stk(X1[:, 256:384], ofs_p[:, 256:384]))
    dxyz_p = jnp.tanh(mm(hpd, Wdp2_ref, bdp2_ref))

    w2 = jnp.concatenate([exp_w, exp_w], axis=0)
    color = col_e * w2 + col_p * (1.0 - w2)
    dattr = attr_e * w2 + attr_p * (1.0 - w2)
    dxyz = dxyz_e * w2 + dxyz_p * (1.0 - w2)

    Rv = R_ref[...]
    scv = scale_ref[...]
    psev = pose_ref[...]
    sp = sp_ref[...]
    rp = rp_ref[...]
    op = op_ref[...]

    ocol_ref[0] = color[:blk]
    ocol_ref[1] = color[blk:]

    for b in range(2):
        sl = slice(b * blk, (b + 1) * blk)
        da = dattr[sl]
        S = scv[b, 0]
        oscl_ref[b] = jnp.exp(sp + da[:, 0:3] * ATTR_SCALE) * S
        oopa_ref[b] = jax.nn.sigmoid(op + da[:, 7:8] * ATTR_SCALE)

        xyz_b = xyz + dxyz[sl] * DEFORM_SCALE
        xs = xyz_b * S
        RbT = RT_ref[...][3 * b:3 * b + 3, :]
        oxyz_ref[b] = (jnp.dot(xs, RbT, preferred_element_type=f32)
                       + psev[b:b + 1, 3:6])

        q = rp + da[:, 3:7] * ATTR_SCALE
        qn = q / jnp.maximum(
            jnp.sqrt(jnp.sum(q * q, axis=1, keepdims=True)), 1e-12)
        r = qn[:, 0:1]; i_ = qn[:, 1:2]; j_ = qn[:, 2:3]; k_ = qn[:, 3:4]
        two_s = 2.0 / jnp.sum(qn * qn, axis=1, keepdims=True)
        M = [[1 - two_s * (j_ * j_ + k_ * k_), two_s * (i_ * j_ - k_ * r),
              two_s * (i_ * k_ + j_ * r)],
             [two_s * (i_ * j_ + k_ * r), 1 - two_s * (i_ * i_ + k_ * k_),
              two_s * (j_ * k_ - i_ * r)],
             [two_s * (i_ * k_ - j_ * r), two_s * (j_ * k_ + i_ * r),
              1 - two_s * (i_ * i_ + j_ * j_)]]
        rm = [[Rv[b, 3 * a_ + 0] * M[0][c_] + Rv[b, 3 * a_ + 1] * M[1][c_]
               + Rv[b, 3 * a_ + 2] * M[2][c_]
               for c_ in range(3)] for a_ in range(3)]
        m00, m01, m02 = rm[0]
        m10, m11, m12 = rm[1]
        m20, m21, m22 = rm[2]
        s0 = 1.0 + m00 + m11 + m22
        s1 = 1.0 + m00 - m11 - m22
        s2 = 1.0 - m00 + m11 - m22
        s3 = 1.0 - m00 - m11 + m22
        qa0 = jnp.sqrt(jnp.maximum(s0, 1e-8))
        qa1 = jnp.sqrt(jnp.maximum(s1, 1e-8))
        qa2 = jnp.sqrt(jnp.maximum(s2, 1e-8))
        qa3 = jnp.sqrt(jnp.maximum(s3, 1e-8))
        cands = [
            [qa0 * qa0, m21 - m12, m02 - m20, m10 - m01],
            [m21 - m12, qa1 * qa1, m10 + m01, m02 + m20],
            [m02 - m20, m10 + m01, qa2 * qa2, m12 + m21],
            [m10 - m01, m20 + m02, m21 + m12, qa3 * qa3],
        ]
        dens = [2.0 * jnp.maximum(qa0, 0.1), 2.0 * jnp.maximum(qa1, 0.1),
                2.0 * jnp.maximum(qa2, 0.1), 2.0 * jnp.maximum(qa3, 0.1)]
        qa = [qa0, qa1, qa2, qa3]
        mx = jnp.maximum(jnp.maximum(qa0, qa1), jnp.maximum(qa2, qa3))
        isel = [(qa_k >= mx).astype(f32) for qa_k in qa]
        f_sel = [isel[0],
                 isel[1] * (1.0 - isel[0]),
                 isel[2] * (1.0 - isel[0]) * (1.0 - isel[1]),
                 isel[3] * (1.0 - isel[0]) * (1.0 - isel[1]) * (1.0 - isel[2])]
        out_cols = []
        for c_ in range(4):
            acc = f_sel[0] * (cands[0][c_] / dens[0])
            for k2 in range(1, 4):
                acc = acc + f_sel[k2] * (cands[k2][c_] / dens[k2])
            out_cols.append(acc)
        orot_ref[b] = jnp.concatenate(out_cols, axis=1)


def kernel(exp_coeff, pose, scale, params, xyz, feature, scales_param,
           rotation_param, opacity_param, landmarks):
    f32 = jnp.float32
    N = xyz.shape[0]
    blk = BLK if N % BLK == 0 else min(N, 8)
    if N % blk != 0:
        raise ValueError("N must be divisible by block size")
    nblk = N // blk

    pc, pa, pd = params["exp_color"], params["exp_attributes"], params["exp_deform"]
    qc, qa_, qd = params["pose_color"], params["pose_attributes"], params["pose_deform"]
    W_feat = jnp.concatenate([pc["w"][0][:FEAT_DIM], pa["w"][0][:FEAT_DIM],
                              qc["w"][0][:FEAT_DIM], qa_["w"][0][:FEAT_DIM]],
                             axis=1)
    W_xyz = jnp.concatenate([pd["w"][0][:27], qd["w"][0][:27]], axis=1)
    W_xyz = jnp.concatenate([W_xyz, jnp.zeros((5, W_xyz.shape[1]), f32)],
                            axis=0)
    Wec_hi = jnp.concatenate([pc["w"][0][FEAT_DIM:], pa["w"][0][FEAT_DIM:],
                              pd["w"][0][27:]], axis=1)
    bec = jnp.concatenate([pc["b"][0], pa["b"][0], pd["b"][0]])[None]
    Wpe_hi = jnp.concatenate([qc["w"][0][FEAT_DIM:], qa_["w"][0][FEAT_DIM:],
                              qd["w"][0][27:]], axis=1)
    bpe = jnp.concatenate([qc["b"][0], qa_["b"][0], qd["b"][0]])[None]

    Rm = _so3_exp(pose[:, :3])
    R9 = Rm.reshape(2, 9)
    RT6 = jnp.concatenate([Rm[0].T, Rm[1].T], axis=0)

    lmkT = landmarks.T
    lmksq = jnp.sum(landmarks * landmarks, axis=1)[None]

    rep = lambda s: pl.BlockSpec(s, lambda i: (0,) * len(s))
    ptr = lambda c: pl.BlockSpec((blk, c), lambda i: (i, 0))
    outr = lambda c: pl.BlockSpec((2, blk, c), lambda i: (0, i, 0))

    b2 = lambda v: v[None]

    operands = [
        xyz, feature, scales_param, rotation_param, opacity_param,
        lmkT, lmksq, exp_coeff, pose, scale, R9, RT6,
        W_feat, W_xyz, Wec_hi, bec, Wpe_hi, bpe,
        pc["w"][1], b2(pc["b"][1]), pa["w"][1], b2(pa["b"][1]),
        pd["w"][1], b2(pd["b"][1]),
        pc["w"][2], b2(pc["b"][2]), pa["w"][2], b2(pa["b"][2]),
        pd["w"][2], b2(pd["b"][2]),
        qc["w"][1], b2(qc["b"][1]), qa_["w"][1], b2(qa_["b"][1]),
        qd["w"][1], b2(qd["b"][1]),
    ]
    in_specs = [
        ptr(3), ptr(FEAT_DIM), ptr(3), ptr(4), ptr(1),
        rep((3, N_LMK)), rep((1, N_LMK)), rep((2, EXP_DIM)),
        rep((2, POSE_DIM)), rep((2, 1)), rep((2, 9)), rep((6, 3)),
        rep((FEAT_DIM, 768)), rep((32, 384)), rep((EXP_DIM, 768)),
        rep((1, 768)), rep((54, 384)), rep((1, 384)),
        rep((256, 256)), rep((1, 256)), rep((256, 256)), rep((1, 256)),
        rep((256, 256)), rep((1, 256)),
        rep((256, 32)), rep((1, 32)), rep((256, 8)), rep((1, 8)),
        rep((256, 3)), rep((1, 3)),
        rep((128, 32)), rep((1, 32)), rep((128, 8)), rep((1, 8)),
        rep((128, 3)), rep((1, 3)),
    ]
    out_shape = [
        jax.ShapeDtypeStruct((2, N, 3), f32),
        jax.ShapeDtypeStruct((2, N, 32), f32),
        jax.ShapeDtypeStruct((2, N, 3), f32),
        jax.ShapeDtypeStruct((2, N, 4), f32),
        jax.ShapeDtypeStruct((2, N, 1), f32),
    ]
    out_specs = [outr(3), outr(32), outr(3), outr(4), outr(1)]

    xyz_out, color, scales, rotation, opacity = pl.pallas_call(
        functools.partial(_body, blk),
        grid=(nblk,),
        in_specs=in_specs,
        out_specs=out_specs,
        out_shape=out_shape,
    )(*operands)
    return xyz_out, color, scales, rotation, opacity


# speedup vs baseline: 4.2373x; 4.2373x over previous
import functools

import jax
import jax.numpy as jnp
from jax.experimental import pallas as pl

FEAT_DIM = 128
EXP_DIM = 64
POSE_DIM = 6
N_LMK = 68
POS_FREQ = 4
NEAR = 0.1
FAR = 0.25
DEFORM_SCALE = 0.3
ATTR_SCALE = 0.05

BLK = 512


def _so3_exp(log_rot, eps=1e-4):
    theta2 = jnp.clip(jnp.sum(log_rot * log_rot, axis=-1), eps)
    theta = jnp.sqrt(theta2)
    fac1 = jnp.sin(theta) / theta
    fac2 = (1.0 - jnp.cos(theta)) / theta2
    x, y, z = log_rot[..., 0], log_rot[..., 1], log_rot[..., 2]
    zz = jnp.zeros_like(x)
    K = jnp.stack([
        jnp.stack([zz, -z, y], axis=-1),
        jnp.stack([z, zz, -x], axis=-1),
        jnp.stack([-y, x, zz], axis=-1),
    ], axis=-2)
    I = jnp.eye(3, dtype=log_rot.dtype)
    return I + fac1[..., None, None] * K + fac2[..., None, None] * (K @ K)


def _pos_embed_rows(xT):
    scaled = jnp.concatenate([xT * (2.0 ** i) for i in range(POS_FREQ)], axis=0)
    return jnp.concatenate([xT, jnp.sin(scaled), jnp.cos(scaled)], axis=0)


def _tmm(w_ref, h, bT_ref):
    out = jax.lax.dot_general(
        w_ref[...], h, (((0,), (1,)), ((), ())),
        preferred_element_type=jnp.float32)
    return out + bT_ref[...]


def _body(blk,
          xyzT_ref, feat_ref, spT_ref, rpT_ref, opT_ref,
          lmk_ref, ec_ref, pose_ref, poseT_ref, scale_ref, R9_ref, R6_ref,
          Wf_ref, Wx_ref, Wech_ref, bech_ref, Wpeh_ref, bpeh_ref,
          Wce2_ref, bce2_ref, Wae2_ref, bae2_ref, Wde2_ref, bde2_ref,
          Wce3_ref, bce3T_ref, Wae3_ref, bae3T_ref, Wde3_ref, bde3T_ref,
          Wcp2_ref, bcp2T_ref, Wap2_ref, bap2T_ref, Wdp2_ref, bdp2T_ref,
          oxyz_ref, ocol_ref, oscl_ref, orot_ref, oopa_ref):
    f32 = jnp.float32
    xyzT = xyzT_ref[...]
    feat = jnp.tanh(feat_ref[...])

    lmk = lmk_ref[...]
    dx = xyzT[0:1, :] - lmk[:, 0:1]
    dy = xyzT[1:2, :] - lmk[:, 1:2]
    dz = xyzT[2:3, :] - lmk[:, 2:3]
    d2 = dx * dx + dy * dy + dz * dz
    dists = jnp.min(d2, axis=0, keepdims=True)
    wT = jnp.clip((FAR - dists) / (FAR - NEAR), 0.0, 1.0)

    embT = _pos_embed_rows(xyzT)
    emb_poseT = _pos_embed_rows(poseT_ref[...])

    F1 = jnp.dot(feat, Wf_ref[...], preferred_element_type=f32)
    X1 = jax.lax.dot_general(
        embT, Wx_ref[...], (((0,), (0,)), ((), ())),
        preferred_element_type=f32)

    ofs_e = jnp.dot(ec_ref[...], Wech_ref[...],
                    preferred_element_type=f32) + bech_ref[...]
    ofs_p = jax.lax.dot_general(
        emb_poseT, Wpeh_ref[...], (((0,), (0,)), ((), ())),
        preferred_element_type=f32) + bpeh_ref[...]

    relu = lambda v: jnp.maximum(v, 0.0)
    mm = lambda a, w_ref, b_ref: (
        jnp.dot(a, w_ref[...], preferred_element_type=f32) + b_ref[...])

    R9 = R9_ref[...]
    R6 = R6_ref[...]
    scv = scale_ref[...]
    poseT = poseT_ref[...]
    spT = spT_ref[...]
    rpT = rpT_ref[...]
    opT = opT_ref[...]

    for b in range(2):
        hc = relu(F1[:, 0:256] + ofs_e[b:b + 1, 0:256])
        colT_e = _tmm(Wce3_ref, relu(mm(hc, Wce2_ref, bce2_ref)),
                      bce3T_ref)
        ha = relu(F1[:, 256:512] + ofs_e[b:b + 1, 256:512])
        attrT_e = _tmm(Wae3_ref, relu(mm(ha, Wae2_ref, bae2_ref)),
                       bae3T_ref)
        hd = relu(X1[:, 0:256] + ofs_e[b:b + 1, 512:768])
        dxyzT_e = jnp.tanh(_tmm(Wde3_ref, relu(mm(hd, Wde2_ref, bde2_ref)),
                                bde3T_ref))
        hpc = relu(F1[:, 512:640] + ofs_p[b:b + 1, 0:128])
        colT_p = _tmm(Wcp2_ref, hpc, bcp2T_ref)
        hpa = relu(F1[:, 640:768] + ofs_p[b:b + 1, 128:256])
        attrT_p = _tmm(Wap2_ref, hpa, bap2T_ref)
        hpd = relu(X1[:, 256:384] + ofs_p[b:b + 1, 256:384])
        dxyzT_p = jnp.tanh(_tmm(Wdp2_ref, hpd, bdp2T_ref))

        pw = 1.0 - wT
        ocol_ref[b] = colT_e * wT + colT_p * pw
        daT = attrT_e * wT + attrT_p * pw
        dxyzT = dxyzT_e * wT + dxyzT_p * pw

        S = scv[b, 0]
        oscl_ref[b] = jnp.exp(spT + daT[0:3] * ATTR_SCALE) * S
        oopa_ref[b] = jax.nn.sigmoid(opT + daT[7:8] * ATTR_SCALE)

        xsT = (xyzT + dxyzT * DEFORM_SCALE) * S
        Rb = R6[3 * b:3 * b + 3, :]
        oxyz_ref[b] = (jnp.dot(Rb, xsT, preferred_element_type=f32)
                       + poseT[3:6, b:b + 1])

        q = rpT + daT[3:7] * ATTR_SCALE
        qn = q / jnp.maximum(
            jnp.sqrt(jnp.sum(q * q, axis=0, keepdims=True)), 1e-12)
        r = qn[0:1]; i_ = qn[1:2]; j_ = qn[2:3]; k_ = qn[3:4]
        two_s = 2.0 / jnp.sum(qn * qn, axis=0, keepdims=True)
        M = [[1 - two_s * (j_ * j_ + k_ * k_), two_s * (i_ * j_ - k_ * r),
              two_s * (i_ * k_ + j_ * r)],
             [two_s * (i_ * j_ + k_ * r), 1 - two_s * (i_ * i_ + k_ * k_),
              two_s * (j_ * k_ - i_ * r)],
             [two_s * (i_ * k_ - j_ * r), two_s * (j_ * k_ + i_ * r),
              1 - two_s * (i_ * i_ + j_ * j_)]]
        rm = [[R9[b, 3 * a_ + 0] * M[0][c_] + R9[b, 3 * a_ + 1] * M[1][c_]
               + R9[b, 3 * a_ + 2] * M[2][c_]
               for c_ in range(3)] for a_ in range(3)]
        m00, m01, m02 = rm[0]
        m10, m11, m12 = rm[1]
        m20, m21, m22 = rm[2]
        s0 = 1.0 + m00 + m11 + m22
        s1 = 1.0 + m00 - m11 - m22
        s2 = 1.0 - m00 + m11 - m22
        s3 = 1.0 - m00 - m11 + m22
        qa = [jnp.sqrt(jnp.maximum(s_, 1e-8)) for s_ in (s0, s1, s2, s3)]
        cands = [
            [qa[0] * qa[0], m21 - m12, m02 - m20, m10 - m01],
            [m21 - m12, qa[1] * qa[1], m10 + m01, m02 + m20],
            [m02 - m20, m10 + m01, qa[2] * qa[2], m12 + m21],
            [m10 - m01, m20 + m02, m21 + m12, qa[3] * qa[3]],
        ]
        mx = jnp.maximum(jnp.maximum(qa[0], qa[1]), jnp.maximum(qa[2], qa[3]))
        isel = [(qa_k >= mx).astype(f32) for qa_k in qa]
        f_sel = [isel[0],
                 isel[1] * (1.0 - isel[0]),
                 isel[2] * (1.0 - isel[0]) * (1.0 - isel[1]),
                 isel[3] * (1.0 - isel[0]) * (1.0 - isel[1]) * (1.0 - isel[2])]
        rows = []
        for c_ in range(4):
            acc = f_sel[0] * cands[0][c_]
            for k2 in range(1, 4):
                acc = acc + f_sel[k2] * cands[k2][c_]
            den = (f_sel[0] * (2.0 * jnp.maximum(qa[0], 0.1))
                   + f_sel[1] * (2.0 * jnp.maximum(qa[1], 0.1))
                   + f_sel[2] * (2.0 * jnp.maximum(qa[2], 0.1))
                   + f_sel[3] * (2.0 * jnp.maximum(qa[3], 0.1)))
            rows.append(acc / den)
        orot_ref[b] = jnp.concatenate(rows, axis=0)


def kernel(exp_coeff, pose, scale, params, xyz, feature, scales_param,
           rotation_param, opacity_param, landmarks):
    f32 = jnp.float32
    N = xyz.shape[0]
    blk = min(BLK, N)
    Np = ((N + blk - 1) // blk) * blk
    nblk = Np // blk

    def padT(a):
        aT = a.T
        if Np != N:
            aT = jnp.concatenate(
                [aT, jnp.zeros((aT.shape[0], Np - N), f32)], axis=1)
        return aT

    def padR(a):
        if Np != N:
            a = jnp.concatenate([a, jnp.zeros((Np - N, a.shape[1]), f32)],
                                axis=0)
        return a

    pc, pa, pd = params["exp_color"], params["exp_attributes"], params["exp_deform"]
    qc, qa_, qd = params["pose_color"], params["pose_attributes"], params["pose_deform"]
    W_feat = jnp.concatenate([pc["w"][0][:FEAT_DIM], pa["w"][0][:FEAT_DIM],
                              qc["w"][0][:FEAT_DIM], qa_["w"][0][:FEAT_DIM]],
                             axis=1)
    perm = ([0, 1, 2] + [3 + 6 * i + j for i in range(POS_FREQ) for j in range(3)]
            + [6 + 6 * i + j for i in range(POS_FREQ) for j in range(3)])
    W_xyz = jnp.concatenate([pd["w"][0][:27][jnp.array(perm)],
                             qd["w"][0][:27][jnp.array(perm)]], axis=1)
    Wec_hi = jnp.concatenate([pc["w"][0][FEAT_DIM:], pa["w"][0][FEAT_DIM:],
                              pd["w"][0][27:]], axis=1)
    bec = jnp.concatenate([pc["b"][0], pa["b"][0], pd["b"][0]])[None]
    perm54 = ([0, 1, 2, 3, 4, 5]
              + [6 + 12 * i + j for i in range(POS_FREQ) for j in range(6)]
              + [12 + 12 * i + j for i in range(POS_FREQ) for j in range(6)])
    Wpe_hi = jnp.concatenate([qc["w"][0][FEAT_DIM:], qa_["w"][0][FEAT_DIM:],
                              qd["w"][0][27:]], axis=1)[jnp.array(perm54)]
    bpe = jnp.concatenate([qc["b"][0], qa_["b"][0], qd["b"][0]])[None]

    Rm = _so3_exp(pose[:, :3])
    R9 = Rm.reshape(2, 9)
    R6 = jnp.concatenate([Rm[0], Rm[1]], axis=0)

    rep = lambda s: pl.BlockSpec(s, lambda i: (0,) * len(s))
    colr = lambda c: pl.BlockSpec((c, blk), lambda i: (0, i))
    outr = lambda c: pl.BlockSpec((2, c, blk), lambda i: (0, 0, i))

    bT = lambda v: v[:, None]

    operands = [
        padT(xyz), padR(feature), padT(scales_param), padT(rotation_param),
        padT(opacity_param),
        landmarks, exp_coeff, pose, pose.T, scale, R9, R6,
        W_feat, W_xyz, Wec_hi, bec, Wpe_hi, bpe,
        pc["w"][1], pc["b"][1][None], pa["w"][1], pa["b"][1][None],
        pd["w"][1], pd["b"][1][None],
        pc["w"][2], bT(pc["b"][2]), pa["w"][2], bT(pa["b"][2]),
        pd["w"][2], bT(pd["b"][2]),
        qc["w"][1], bT(qc["b"][1]), qa_["w"][1], bT(qa_["b"][1]),
        qd["w"][1], bT(qd["b"][1]),
    ]
    in_specs = [
        colr(3), pl.BlockSpec((blk, FEAT_DIM), lambda i: (i, 0)),
        colr(3), colr(4), colr(1),
        rep((N_LMK, 3)), rep((2, EXP_DIM)), rep((2, POSE_DIM)),
        rep((POSE_DIM, 2)), rep((2, 1)), rep((2, 9)), rep((6, 3)),
        rep((FEAT_DIM, 768)), rep((27, 384)), rep((EXP_DIM, 768)),
        rep((1, 768)), rep((54, 384)), rep((1, 384)),
        rep((256, 256)), rep((1, 256)), rep((256, 256)), rep((1, 256)),
        rep((256, 256)), rep((1, 256)),
        rep((256, 32)), rep((32, 1)), rep((256, 8)), rep((8, 1)),
        rep((256, 3)), rep((3, 1)),
        rep((128, 32)), rep((32, 1)), rep((128, 8)), rep((8, 1)),
        rep((128, 3)), rep((3, 1)),
    ]
    out_shape = [
        jax.ShapeDtypeStruct((2, 3, Np), f32),
        jax.ShapeDtypeStruct((2, 32, Np), f32),
        jax.ShapeDtypeStruct((2, 3, Np), f32),
        jax.ShapeDtypeStruct((2, 4, Np), f32),
        jax.ShapeDtypeStruct((2, 1, Np), f32),
    ]
    out_specs = [outr(3), outr(32), outr(3), outr(4), outr(1)]

    outs = pl.pallas_call(
        functools.partial(_body, blk),
        grid=(nblk,),
        in_specs=in_specs,
        out_specs=out_specs,
        out_shape=out_shape,
    )(*operands)
    xyz_out, color, scales, rotation, opacity = (
        jnp.swapaxes(o, 1, 2)[:, :N] for o in outs)
    return xyz_out, color, scales, rotation, opacity
